# baseline (speedup 1.0000x reference)
import functools

import jax
import jax.numpy as jnp
from jax import lax
from jax.experimental import pallas as pl
from jax.experimental.pallas import tpu as pltpu
from jax.experimental.pallas import tpu_sc as plsc

VOCAB = 1000
N_TOK = 1024 * 50

_info = plsc.get_sparse_core_info()
NC, NS = _info.num_cores, _info.num_subcores
NW = NC * NS
PER_TILE = N_TOK // NW
CH = 32
NCHUNK = PER_TILE // CH
_PICK_SLICES = [(k * 128, 128) for k in range(PER_TILE // 128)]
if PER_TILE % 128:
    _PICK_SLICES.append((PER_TILE - PER_TILE % 128, PER_TILE % 128))


def _lse_body(table_ref, lse_ref, tcopy_ref):
    t = table_ref[...]
    m = jnp.max(t, axis=1, keepdims=True)
    s = jnp.sum(jnp.exp(t - m), axis=1, keepdims=True)
    lse_ref[...] = m + jnp.log(s)
    tcopy_ref[...] = t


def _lse_rows(table):
    return pl.pallas_call(
        _lse_body,
        out_shape=[
            jax.ShapeDtypeStruct((VOCAB, 1), jnp.float32),
            jax.ShapeDtypeStruct((VOCAB, VOCAB), jnp.float32),
        ],
    )(table)


def _sc_body(idx_hbm, tgt_hbm, table_hbm, tflat_hbm, lse_hbm,
             out_hbm, part_hbm,
             idx_v, tgt_v, fidx_v, vals_v, lsec_v, rows_v, acc_v,
             sem_g0, sem_g1, sem_s0, sem_s1, sem_t):
    wid = lax.axis_index("s") * NC + lax.axis_index("c")
    base = wid * PER_TILE

    pltpu.sync_copy(idx_hbm.at[pl.ds(base, PER_TILE)], idx_v)
    pltpu.sync_copy(tgt_hbm.at[pl.ds(base, PER_TILE)], tgt_v)
    acc_v[...] = jnp.zeros((16,), jnp.float32)

    def fidx_step(j, _):
        o = pl.multiple_of(j * 16, 16)
        i16 = idx_v[pl.ds(o, 16)]
        t16 = tgt_v[pl.ds(o, 16)]
        fidx_v[pl.ds(o, 16)] = i16 * VOCAB + t16
        return 0

    lax.fori_loop(0, PER_TILE // 16, fidx_step, 0)

    def tiny_copies():
        for (o, n) in _PICK_SLICES:
            yield pltpu.make_async_copy(
                tflat_hbm.at[fidx_v.at[pl.ds(o, n)]],
                vals_v.at[pl.ds(o, n)], sem_t)
            yield pltpu.make_async_copy(
                lse_hbm.at[idx_v.at[pl.ds(o, n)]],
                lsec_v.at[pl.ds(o, n)], sem_t)

    for cp in tiny_copies():
        cp.start()

    sem_g = (sem_g0, sem_g1)
    sem_s = (sem_s0, sem_s1)

    def g_copy(c, b):
        o = pl.multiple_of(c * CH, CH)
        return pltpu.make_async_copy(
            table_hbm.at[idx_v.at[pl.ds(o, CH)]], rows_v.at[b], sem_g[b])

    def s_copy(c, b):
        o = pl.multiple_of(base + c * CH, CH)
        return pltpu.make_async_copy(
            rows_v.at[b], out_hbm.at[pl.ds(o, CH)], sem_s[b])

    g_copy(0, 0).start()
    g_copy(1, 1).start()

    def pair(g, _):
        c0 = g * 2
        g_copy(c0, 0).wait()
        s_copy(c0, 0).start()
        g_copy(c0 + 1, 1).wait()
        s_copy(c0 + 1, 1).start()

        @pl.when(g < NCHUNK // 2 - 1)
        def _prefetch():
            s_copy(c0, 0).wait()
            g_copy(c0 + 2, 0).start()
            s_copy(c0 + 1, 1).wait()
            g_copy(c0 + 3, 1).start()

        return 0

    lax.fori_loop(0, NCHUNK // 2, pair, 0)
    s_copy(NCHUNK - 2, 0).wait()
    s_copy(NCHUNK - 1, 1).wait()

    for cp in tiny_copies():
        cp.wait()

    def loss_step(j, _):
        o = pl.multiple_of(j * 16, 16)
        acc_v[...] = acc_v[...] + lsec_v[pl.ds(o, 16)] - vals_v[pl.ds(o, 16)]
        return 0

    lax.fori_loop(0, PER_TILE // 16, loss_step, 0)
    pltpu.sync_copy(acc_v, part_hbm.at[wid])


_sc_call = functools.partial(
    pl.kernel,
    out_type=[
        jax.ShapeDtypeStruct((N_TOK, VOCAB), jnp.float32),
        jax.ShapeDtypeStruct((NW, 16), jnp.float32),
    ],
    mesh=plsc.VectorSubcoreMesh(core_axis_name="c", subcore_axis_name="s"),
    compiler_params=pltpu.CompilerParams(use_tc_tiling_on_sc=False),
    scratch_types=[
        pltpu.VMEM((PER_TILE,), jnp.int32),
        pltpu.VMEM((PER_TILE,), jnp.int32),
        pltpu.VMEM((PER_TILE,), jnp.int32),
        pltpu.VMEM((PER_TILE,), jnp.float32),
        pltpu.VMEM((PER_TILE,), jnp.float32),
        pltpu.VMEM((2, CH, VOCAB), jnp.float32),
        pltpu.VMEM((16,), jnp.float32),
        pltpu.SemaphoreType.DMA,
        pltpu.SemaphoreType.DMA,
        pltpu.SemaphoreType.DMA,
        pltpu.SemaphoreType.DMA,
        pltpu.SemaphoreType.DMA,
    ],
)(_sc_body)


def _loss_body(part_ref, loss_ref):
    loss_ref[...] = jnp.sum(part_ref[...]).reshape(1, 1) / N_TOK


def _loss_reduce(partials):
    return pl.pallas_call(
        _loss_body,
        out_shape=jax.ShapeDtypeStruct((1, 1), jnp.float32),
    )(partials)


@jax.jit
def kernel(idx, targets, table):
    B, T = idx.shape
    idx_f = idx.reshape(N_TOK).astype(jnp.int32)
    tgt_f = targets.reshape(N_TOK).astype(jnp.int32)
    lse, tcopy = _lse_rows(table)
    logits_flat, partials = _sc_call(
        idx_f, tgt_f, table, tcopy.reshape(VOCAB * VOCAB), lse.reshape(VOCAB))
    loss = _loss_reduce(partials)
    return logits_flat.reshape(B, T, VOCAB), loss[0, 0]
